# Initial kernel scaffold; baseline (speedup 1.0000x reference)
import jax
import jax.numpy as jnp
from jax.experimental import pallas as pl


def kernel(x, edge_index, params):
    raise NotImplementedError("write your pallas kernel here")


# speedup vs baseline: 38.0051x; 38.0051x over previous
import functools

import jax
import jax.numpy as jnp
from jax import lax
from jax.experimental import pallas as pl
from jax.experimental.pallas import tpu as pltpu
from jax.experimental.pallas import tpu_sc as plsc

N = 50000
HID = 20
NEG = 0.2
ROWW = 32
ACCW = 24
NW = 32
EPW = 50176
EPAD = NW * EPW
CHUNK = 128
NCHUNK = EPW // CHUNK
NSLOT = 4
RACC = 51200
RPT = RACC // 16
DCH = 80
BIN = N
BLK = 1000
GRID = N // BLK
PAD0 = ROWW - HID - 1


def _elu(z):
    return jnp.where(z > 0, z, jnp.exp(jnp.minimum(z, 0.0)) - 1.0)


def _combine(p_ref, linp_ref, cbp_ref):
    num = p_ref[0, :, 0:HID] + p_ref[1, :, 0:HID]
    den = p_ref[0, :, HID:HID + 1] + p_ref[1, :, HID:HID + 1]
    conv = num / (den + 1e-30) + cbp_ref[...]
    return _elu(conv + linp_ref[...])


def _dense_common(h, W_ref, as_ref, ad_ref, lW_ref, lb_ref,
                  table_ref, dalpha_ref, amax_ref, lin_ref):
    xw = jnp.dot(h, W_ref[...], preferred_element_type=jnp.float32)
    asrc = jnp.dot(xw, as_ref[...], preferred_element_type=jnp.float32)
    adst = jnp.dot(xw, ad_ref[...], preferred_element_type=jnp.float32)
    table_ref[...] = jnp.concatenate(
        [xw, asrc, jnp.zeros((xw.shape[0], PAD0), jnp.float32)], axis=1)
    dalpha_ref[...] = adst
    lin_ref[...] = jnp.dot(h, lW_ref[...], preferred_element_type=jnp.float32) + lb_ref[...]
    m = jnp.max(asrc)

    @pl.when(pl.program_id(0) == 0)
    def _():
        amax_ref[...] = jnp.full((8, 128), -jnp.inf, jnp.float32)

    amax_ref[...] = jnp.maximum(amax_ref[...], m)


def _dense0_body(x_ref, W_ref, as_ref, ad_ref, lW_ref, lb_ref,
                 table_ref, dalpha_ref, amax_ref, lin_ref):
    _dense_common(x_ref[...], W_ref, as_ref, ad_ref, lW_ref, lb_ref,
                  table_ref, dalpha_ref, amax_ref, lin_ref)


def _densek_body(p_ref, linp_ref, cbp_ref, W_ref, as_ref, ad_ref, lW_ref, lb_ref,
                 table_ref, dalpha_ref, amax_ref, lin_ref):
    h = _combine(p_ref, linp_ref, cbp_ref)
    _dense_common(h, W_ref, as_ref, ad_ref, lW_ref, lb_ref,
                  table_ref, dalpha_ref, amax_ref, lin_ref)


def _mlp_body(p_ref, linp_ref, cbp_ref,
              w1, b1, w2, b2, w3, b3, w4, b4, out_ref):
    z = _combine(p_ref, linp_ref, cbp_ref)
    for w, b in ((w1, b1), (w2, b2), (w3, b3), (w4, b4)):
        z = _elu(jnp.dot(z, w[...], preferred_element_type=jnp.float32) + b[...])
    out_ref[...] = 1.0 / (1.0 + jnp.exp(-z))


def _full_spec(shape):
    nd = len(shape)
    return pl.BlockSpec(shape, lambda b, _n=nd: (0,) * _n)


_DENSE_OUT = [
    jax.ShapeDtypeStruct((N, ROWW), jnp.float32),
    jax.ShapeDtypeStruct((N, 1), jnp.float32),
    jax.ShapeDtypeStruct((8, 128), jnp.float32),
    jax.ShapeDtypeStruct((N, HID), jnp.float32),
]
_DENSE_OUT_SPECS = [
    pl.BlockSpec((BLK, ROWW), lambda b: (b, 0)),
    pl.BlockSpec((BLK, 1), lambda b: (b, 0)),
    pl.BlockSpec((8, 128), lambda b: (0, 0)),
    pl.BlockSpec((BLK, HID), lambda b: (b, 0)),
]


def _dense0(x, W, a_s, a_d, lW, lb):
    fin = x.shape[1]
    return pl.pallas_call(
        _dense0_body,
        grid=(GRID,),
        in_specs=[
            pl.BlockSpec((BLK, fin), lambda b: (b, 0)),
            _full_spec((fin, HID)),
            _full_spec((HID, 1)),
            _full_spec((HID, 1)),
            _full_spec((fin, HID)),
            _full_spec((1, HID)),
        ],
        out_specs=_DENSE_OUT_SPECS,
        out_shape=_DENSE_OUT,
    )(x, W, a_s, a_d, lW, lb)


def _densek(partial, lin_prev, cb_prev, W, a_s, a_d, lW, lb):
    return pl.pallas_call(
        _densek_body,
        grid=(GRID,),
        in_specs=[
            pl.BlockSpec((2, BLK, ACCW), lambda b: (0, b, 0)),
            pl.BlockSpec((BLK, HID), lambda b: (b, 0)),
            _full_spec((1, HID)),
            _full_spec((HID, HID)),
            _full_spec((HID, 1)),
            _full_spec((HID, 1)),
            _full_spec((HID, HID)),
            _full_spec((1, HID)),
        ],
        out_specs=_DENSE_OUT_SPECS,
        out_shape=_DENSE_OUT,
    )(partial, lin_prev, cb_prev, W, a_s, a_d, lW, lb)


def _mlp(partial, lin_prev, cb_prev, fWs, fbs):
    wspecs = []
    args = []
    for w, b in zip(fWs, fbs):
        wspecs += [_full_spec(w.shape), _full_spec((1,) + b.shape)]
        args += [w, b.reshape(1, -1)]
    return pl.pallas_call(
        _mlp_body,
        grid=(GRID,),
        in_specs=[
            pl.BlockSpec((2, BLK, ACCW), lambda b: (0, b, 0)),
            pl.BlockSpec((BLK, HID), lambda b: (b, 0)),
            _full_spec((1, HID)),
        ] + wspecs,
        out_specs=pl.BlockSpec((BLK, 1), lambda b: (b, 0)),
        out_shape=jax.ShapeDtypeStruct((N, 1), jnp.float32),
    )(partial, lin_prev, cb_prev, *args)


def _edge_body(table_h, eidx_h, dal_h, amx_h, out_h,
               acc,
               idx0, idx1, idx2, idx3,
               sdx0, sdx1, sdx2, sdx3,
               rows0, rows1, rows2, rows3,
               ad0, ad1, ad2, ad3,
               or0, or1, or2, or3,
               amx,
               g0, g1, g2, g3, a0, a1, a2, a3, s0, s1, s2, s3):
    cid = lax.axis_index("c")
    sid = lax.axis_index("s")
    wid = sid * 2 + cid

    idxs = (idx0, idx1, idx2, idx3)
    sdxs = (sdx0, sdx1, sdx2, sdx3)
    rowss = (rows0, rows1, rows2, rows3)
    ads = (ad0, ad1, ad2, ad3)
    ors = (or0, or1, or2, or3)
    gsem = (g0, g1, g2, g3)
    asem = (a0, a1, a2, a3)
    ssem = (s0, s1, s2, s3)

    pltpu.sync_copy(amx_h, amx)

    zero16 = jnp.zeros((16,), jnp.float32)

    for ob in ors:
        def zrow(r, c, _ob=ob):
            _ob[r, pl.ds(0, 16)] = zero16
            _ob[r, pl.ds(8, 16)] = zero16
            return c
        lax.fori_loop(0, CHUNK, zrow, 0)

    def zacc(i, c):
        pltpu.sync_copy(or0.at[pl.ds(0, DCH)],
                        acc.at[pl.ds(sid * RPT + i * DCH, DCH)])
        return c
    lax.fori_loop(0, RPT // DCH, zacc, 0)
    plsc.subcore_barrier()

    iota16 = lax.iota(jnp.int32, 16)
    amax_v = amx[...]
    col20 = jnp.full((16,), HID, jnp.int32)

    def issue(k, slot):
        pltpu.sync_copy(eidx_h.at[wid, k], idxs[slot])
        pltpu.async_copy(table_h.at[idxs[slot].at[0]], rowss[slot], gsem[slot])
        pltpu.async_copy(dal_h.at[idxs[slot].at[1]], ads[slot], asem[slot])

    def compute(slot):
        rows = rowss[slot]
        orows = ors[slot]
        adbuf = ads[slot]
        idxb = idxs[slot]
        sdx = sdxs[slot]
        for g in range(CHUNK // 16):
            row_ids = g * 16 + iota16
            sdx[pl.ds(g * 16, 16)] = idxb[1, pl.ds(g * 16, 16)]
            a_d = adbuf[pl.ds(g * 16, 16)]
            t2 = amax_v + a_d
            shift = jnp.maximum(t2, NEG * t2)
            a_s = plsc.load_gather(rows, [row_ids, col20])
            t = a_s + a_d
            val = jnp.maximum(t, NEG * t)
            e = jnp.exp(val - shift)
            plsc.store_scatter(orows, [row_ids, col20], e)
            for f in range(HID):
                colf = jnp.full((16,), f, jnp.int32)
                v = plsc.load_gather(rows, [row_ids, colf])
                plsc.store_scatter(orows, [row_ids, colf], v * e)

    def drain_g(slot):
        pltpu.make_async_copy(table_h.at[pl.ds(0, CHUNK)],
                              rowss[slot], gsem[slot]).wait()

    def drain_a(slot):
        pltpu.make_async_copy(dal_h.at[pl.ds(0, CHUNK)],
                              ads[slot], asem[slot]).wait()

    def drain_s(slot):
        pltpu.make_async_copy(out_h.at[cid, pl.ds(0, CHUNK)],
                              ors[slot], ssem[slot]).wait()

    for k in range(NSLOT - 1):
        issue(k, k)

    def body(j, c):
        for u in range(NSLOT):
            k = NSLOT * j + u
            drain_g(u)
            drain_a(u)

            @pl.when(k + NSLOT - 1 < NCHUNK)
            def _(_u=u, _k=k):
                issue(_k + NSLOT - 1, (_u + NSLOT - 1) % NSLOT)

            @pl.when(j > 0)
            def _(_u=u):
                drain_s(_u)
            compute(u)
            pltpu.async_copy(ors[u], acc.at[sdxs[u]], ssem[u], add=True)
        return c
    lax.fori_loop(0, NCHUNK // NSLOT, body, 0)

    for u in range(NSLOT):
        drain_s(u)
    plsc.subcore_barrier()

    def dump(i, c):
        r0 = sid * RPT + i * DCH

        @pl.when(r0 < N)
        def _():
            pltpu.sync_copy(acc.at[pl.ds(r0, DCH)], or0.at[pl.ds(0, DCH)])
            pltpu.sync_copy(or0.at[pl.ds(0, DCH)], out_h.at[cid, pl.ds(r0, DCH)])
        return c
    lax.fori_loop(0, RPT // DCH, dump, 0)


def _edge_pass(table, eidx, dalpha, amax16):
    kern = pl.kernel(
        _edge_body,
        out_type=jax.ShapeDtypeStruct((2, N, ACCW), jnp.float32),
        mesh=plsc.VectorSubcoreMesh(core_axis_name="c", subcore_axis_name="s"),
        compiler_params=pltpu.CompilerParams(
            needs_layout_passes=False, use_tc_tiling_on_sc=False),
        scratch_types=(
            [pltpu.VMEM_SHARED((RACC, ACCW), jnp.float32)]
            + [pltpu.VMEM((2, CHUNK), jnp.int32) for _ in range(NSLOT)]
            + [pltpu.VMEM((CHUNK,), jnp.int32) for _ in range(NSLOT)]
            + [pltpu.VMEM((CHUNK, ROWW), jnp.float32) for _ in range(NSLOT)]
            + [pltpu.VMEM((CHUNK,), jnp.float32) for _ in range(NSLOT)]
            + [pltpu.VMEM((CHUNK, ACCW), jnp.float32) for _ in range(NSLOT)]
            + [pltpu.VMEM((16,), jnp.float32)]
            + [pltpu.SemaphoreType.DMA for _ in range(3 * NSLOT)]
        ),
    )
    return kern(table, eidx, dalpha, amax16)


def kernel(x, edge_index, params):
    src = edge_index[0]
    dst = edge_index[1]
    pad = EPAD - src.shape[0]
    srcs = jnp.concatenate([src, jnp.zeros((pad,), jnp.int32)])
    dsts = jnp.concatenate([dst, jnp.full((pad,), BIN, jnp.int32)])
    eidx = jnp.stack([srcs.reshape(NW, NCHUNK, CHUNK),
                      dsts.reshape(NW, NCHUNK, CHUNK)], axis=2)

    partial = None
    lin = None
    for i in range(4):
        W = params["cW"][i]
        a_s = params["cas"][i].reshape(HID, 1)
        a_d = params["cad"][i].reshape(HID, 1)
        lW = params["lW"][i]
        lb = params["lb"][i].reshape(1, HID)
        if i == 0:
            table, dalpha, amax, lin = _dense0(x, W, a_s, a_d, lW, lb)
        else:
            cbp = params["cb"][i - 1].reshape(1, HID)
            table, dalpha, amax, lin = _densek(
                partial, lin, cbp, W, a_s, a_d, lW, lb)
        partial = _edge_pass(table, eidx, dalpha.reshape(N), amax[0, 0:16])

    out = _mlp(partial, lin, params["cb"][3].reshape(1, HID),
               params["fW"], params["fb"])
    return out.reshape(N)


# speedup vs baseline: 53.9258x; 1.4189x over previous
import functools

import jax
import jax.numpy as jnp
from jax import lax
from jax.experimental import pallas as pl
from jax.experimental.pallas import tpu as pltpu
from jax.experimental.pallas import tpu_sc as plsc

N = 50000
HID = 20
NEG = 0.2
ROWW = 24
ACCW = 24
NW = 32
EPW = 50176
EPAD = NW * EPW
CHUNK = 128
NCHUNK = EPW // CHUNK
NSLOT = 4
RACC = 51200
RPT = RACC // 16
DCH = 80
BIN = N
BLK = 1000
GRID = N // BLK
PAD0 = ROWW - HID - 1


def _elu(z):
    return jnp.where(z > 0, z, jnp.exp(jnp.minimum(z, 0.0)) - 1.0)


def _combine(p_ref, linp_ref, cbp_ref):
    num = p_ref[0, :, 0:HID] + p_ref[1, :, 0:HID]
    den = p_ref[0, :, HID:HID + 1] + p_ref[1, :, HID:HID + 1]
    conv = num / (den + 1e-30) + cbp_ref[...]
    return _elu(conv + linp_ref[...])


def _dense_common(h, W_ref, as_ref, ad_ref, lW_ref, lb_ref,
                  table_ref, dalpha_ref, amax_ref, lin_ref):
    xw = jnp.dot(h, W_ref[...], preferred_element_type=jnp.float32)
    asrc = jnp.dot(xw, as_ref[...], preferred_element_type=jnp.float32)
    adst = jnp.dot(xw, ad_ref[...], preferred_element_type=jnp.float32)
    table_ref[...] = jnp.concatenate(
        [xw, asrc, jnp.zeros((xw.shape[0], PAD0), jnp.float32)], axis=1)
    dalpha_ref[...] = adst
    lin_ref[...] = jnp.dot(h, lW_ref[...], preferred_element_type=jnp.float32) + lb_ref[...]
    m = jnp.max(asrc)

    @pl.when(pl.program_id(0) == 0)
    def _():
        amax_ref[...] = jnp.full((8, 128), -jnp.inf, jnp.float32)

    amax_ref[...] = jnp.maximum(amax_ref[...], m)


def _dense0_body(x_ref, W_ref, as_ref, ad_ref, lW_ref, lb_ref,
                 table_ref, dalpha_ref, amax_ref, lin_ref):
    _dense_common(x_ref[...], W_ref, as_ref, ad_ref, lW_ref, lb_ref,
                  table_ref, dalpha_ref, amax_ref, lin_ref)


def _densek_body(p_ref, linp_ref, cbp_ref, W_ref, as_ref, ad_ref, lW_ref, lb_ref,
                 table_ref, dalpha_ref, amax_ref, lin_ref):
    h = _combine(p_ref, linp_ref, cbp_ref)
    _dense_common(h, W_ref, as_ref, ad_ref, lW_ref, lb_ref,
                  table_ref, dalpha_ref, amax_ref, lin_ref)


def _mlp_body(p_ref, linp_ref, cbp_ref,
              w1, b1, w2, b2, w3, b3, w4, b4, out_ref):
    z = _combine(p_ref, linp_ref, cbp_ref)
    for w, b in ((w1, b1), (w2, b2), (w3, b3), (w4, b4)):
        z = _elu(jnp.dot(z, w[...], preferred_element_type=jnp.float32) + b[...])
    out_ref[...] = 1.0 / (1.0 + jnp.exp(-z))


def _full_spec(shape):
    nd = len(shape)
    return pl.BlockSpec(shape, lambda b, _n=nd: (0,) * _n)


_DENSE_OUT = [
    jax.ShapeDtypeStruct((N, ROWW), jnp.float32),
    jax.ShapeDtypeStruct((N, 1), jnp.float32),
    jax.ShapeDtypeStruct((8, 128), jnp.float32),
    jax.ShapeDtypeStruct((N, HID), jnp.float32),
]
_DENSE_OUT_SPECS = [
    pl.BlockSpec((BLK, ROWW), lambda b: (b, 0)),
    pl.BlockSpec((BLK, 1), lambda b: (b, 0)),
    pl.BlockSpec((8, 128), lambda b: (0, 0)),
    pl.BlockSpec((BLK, HID), lambda b: (b, 0)),
]


def _dense0(x, W, a_s, a_d, lW, lb):
    fin = x.shape[1]
    return pl.pallas_call(
        _dense0_body,
        grid=(GRID,),
        in_specs=[
            pl.BlockSpec((BLK, fin), lambda b: (b, 0)),
            _full_spec((fin, HID)),
            _full_spec((HID, 1)),
            _full_spec((HID, 1)),
            _full_spec((fin, HID)),
            _full_spec((1, HID)),
        ],
        out_specs=_DENSE_OUT_SPECS,
        out_shape=_DENSE_OUT,
    )(x, W, a_s, a_d, lW, lb)


def _densek(partial, lin_prev, cb_prev, W, a_s, a_d, lW, lb):
    return pl.pallas_call(
        _densek_body,
        grid=(GRID,),
        in_specs=[
            pl.BlockSpec((2, BLK, ACCW), lambda b: (0, b, 0)),
            pl.BlockSpec((BLK, HID), lambda b: (b, 0)),
            _full_spec((1, HID)),
            _full_spec((HID, HID)),
            _full_spec((HID, 1)),
            _full_spec((HID, 1)),
            _full_spec((HID, HID)),
            _full_spec((1, HID)),
        ],
        out_specs=_DENSE_OUT_SPECS,
        out_shape=_DENSE_OUT,
    )(partial, lin_prev, cb_prev, W, a_s, a_d, lW, lb)


def _mlp(partial, lin_prev, cb_prev, fWs, fbs):
    wspecs = []
    args = []
    for w, b in zip(fWs, fbs):
        wspecs += [_full_spec(w.shape), _full_spec((1,) + b.shape)]
        args += [w, b.reshape(1, -1)]
    return pl.pallas_call(
        _mlp_body,
        grid=(GRID,),
        in_specs=[
            pl.BlockSpec((2, BLK, ACCW), lambda b: (0, b, 0)),
            pl.BlockSpec((BLK, HID), lambda b: (b, 0)),
            _full_spec((1, HID)),
        ] + wspecs,
        out_specs=pl.BlockSpec((BLK, 1), lambda b: (b, 0)),
        out_shape=jax.ShapeDtypeStruct((N, 1), jnp.float32),
    )(partial, lin_prev, cb_prev, *args)


def _edge_body(table_h, eidx_h, dal_h, amx_h, out_h,
               acc,
               idx0, idx1, idx2, idx3,
               sdx0, sdx1, sdx2, sdx3,
               rows0, rows1, rows2, rows3,
               ad0, ad1, ad2, ad3,
               or0, or1, or2, or3,
               amx,
               g0, g1, g2, g3, a0, a1, a2, a3, s0, s1, s2, s3):
    cid = lax.axis_index("c")
    sid = lax.axis_index("s")
    wid = sid * 2 + cid

    idxs = (idx0, idx1, idx2, idx3)
    sdxs = (sdx0, sdx1, sdx2, sdx3)
    rowss = (rows0, rows1, rows2, rows3)
    ads = (ad0, ad1, ad2, ad3)
    ors = (or0, or1, or2, or3)
    gsem = (g0, g1, g2, g3)
    asem = (a0, a1, a2, a3)
    ssem = (s0, s1, s2, s3)

    pltpu.sync_copy(amx_h, amx)

    zero16 = jnp.zeros((16,), jnp.float32)

    for ob in ors:
        def zrow(r, c, _ob=ob):
            _ob[r, pl.ds(0, 16)] = zero16
            _ob[r, pl.ds(8, 16)] = zero16
            return c
        lax.fori_loop(0, CHUNK, zrow, 0)

    def zacc(i, c):
        pltpu.sync_copy(or0.at[pl.ds(0, DCH)],
                        acc.at[pl.ds(sid * RPT + i * DCH, DCH)])
        return c
    lax.fori_loop(0, RPT // DCH, zacc, 0)
    plsc.subcore_barrier()

    iota16 = lax.iota(jnp.int32, 16)
    amax_v = amx[...]
    col20 = jnp.full((16,), HID, jnp.int32)

    def issue(k, slot):
        pltpu.sync_copy(eidx_h.at[wid, k], idxs[slot])
        pltpu.async_copy(table_h.at[idxs[slot].at[0]], rowss[slot], gsem[slot])
        pltpu.async_copy(dal_h.at[idxs[slot].at[1]], ads[slot], asem[slot])

    def compute(slot):
        rows = rowss[slot]
        orows = ors[slot]
        adbuf = ads[slot]
        idxb = idxs[slot]
        sdx = sdxs[slot]
        for g in range(CHUNK // 16):
            row_ids = g * 16 + iota16
            sdx[pl.ds(g * 16, 16)] = idxb[1, pl.ds(g * 16, 16)]
            a_d = adbuf[pl.ds(g * 16, 16)]
            t2 = amax_v + a_d
            shift = jnp.maximum(t2, NEG * t2)
            a_s = plsc.load_gather(rows, [row_ids, col20])
            t = a_s + a_d
            val = jnp.maximum(t, NEG * t)
            e = jnp.exp(val - shift)
            plsc.store_scatter(orows, [row_ids, col20], e)
            for f in range(HID):
                colf = jnp.full((16,), f, jnp.int32)
                v = plsc.load_gather(rows, [row_ids, colf])
                plsc.store_scatter(orows, [row_ids, colf], v * e)

    def drain_g(slot):
        pltpu.make_async_copy(table_h.at[pl.ds(0, CHUNK)],
                              rowss[slot], gsem[slot]).wait()

    def drain_a(slot):
        pltpu.make_async_copy(dal_h.at[pl.ds(0, CHUNK)],
                              ads[slot], asem[slot]).wait()

    def drain_s(slot):
        pltpu.make_async_copy(out_h.at[cid, pl.ds(0, CHUNK)],
                              ors[slot], ssem[slot]).wait()

    for k in range(NSLOT - 1):
        issue(k, k)

    def body(j, c):
        for u in range(NSLOT):
            k = NSLOT * j + u
            drain_g(u)
            drain_a(u)

            @pl.when(k + NSLOT - 1 < NCHUNK)
            def _(_u=u, _k=k):
                issue(_k + NSLOT - 1, (_u + NSLOT - 1) % NSLOT)

            @pl.when(j > 0)
            def _(_u=u):
                drain_s(_u)
            compute(u)
            pltpu.async_copy(ors[u], acc.at[sdxs[u]], ssem[u], add=True)
        return c
    lax.fori_loop(0, NCHUNK // NSLOT, body, 0)

    for u in range(NSLOT):
        drain_s(u)
    plsc.subcore_barrier()

    def dump(i, c):
        r0 = sid * RPT + i * DCH

        @pl.when(r0 < N)
        def _():
            pltpu.sync_copy(acc.at[pl.ds(r0, DCH)], or0.at[pl.ds(0, DCH)])
            pltpu.sync_copy(or0.at[pl.ds(0, DCH)], out_h.at[cid, pl.ds(r0, DCH)])
        return c
    lax.fori_loop(0, RPT // DCH, dump, 0)


def _edge_pass(table, eidx, dalpha, amax16):
    kern = pl.kernel(
        _edge_body,
        out_type=jax.ShapeDtypeStruct((2, N, ACCW), jnp.float32),
        mesh=plsc.VectorSubcoreMesh(core_axis_name="c", subcore_axis_name="s"),
        compiler_params=pltpu.CompilerParams(
            needs_layout_passes=False, use_tc_tiling_on_sc=False),
        scratch_types=(
            [pltpu.VMEM_SHARED((RACC, ACCW), jnp.float32)]
            + [pltpu.VMEM((2, CHUNK), jnp.int32) for _ in range(NSLOT)]
            + [pltpu.VMEM((CHUNK,), jnp.int32) for _ in range(NSLOT)]
            + [pltpu.VMEM((CHUNK, ROWW), jnp.float32) for _ in range(NSLOT)]
            + [pltpu.VMEM((CHUNK,), jnp.float32) for _ in range(NSLOT)]
            + [pltpu.VMEM((CHUNK, ACCW), jnp.float32) for _ in range(NSLOT)]
            + [pltpu.VMEM((16,), jnp.float32)]
            + [pltpu.SemaphoreType.DMA for _ in range(3 * NSLOT)]
        ),
    )
    return kern(table, eidx, dalpha, amax16)


def kernel(x, edge_index, params):
    src = edge_index[0]
    dst = edge_index[1]
    pad = EPAD - src.shape[0]
    srcs = jnp.concatenate([src, jnp.zeros((pad,), jnp.int32)])
    dsts = jnp.concatenate([dst, jnp.full((pad,), BIN, jnp.int32)])
    eidx = jnp.stack([srcs.reshape(NW, NCHUNK, CHUNK),
                      dsts.reshape(NW, NCHUNK, CHUNK)], axis=2)

    partial = None
    lin = None
    for i in range(4):
        W = params["cW"][i]
        a_s = params["cas"][i].reshape(HID, 1)
        a_d = params["cad"][i].reshape(HID, 1)
        lW = params["lW"][i]
        lb = params["lb"][i].reshape(1, HID)
        if i == 0:
            table, dalpha, amax, lin = _dense0(x, W, a_s, a_d, lW, lb)
        else:
            cbp = params["cb"][i - 1].reshape(1, HID)
            table, dalpha, amax, lin = _densek(
                partial, lin, cbp, W, a_s, a_d, lW, lb)
        partial = _edge_pass(table, eidx, dalpha.reshape(N), amax[0, 0:16])

    out = _mlp(partial, lin, params["cb"][3].reshape(1, HID),
               params["fW"], params["fb"])
    return out.reshape(N)


# speedup vs baseline: 53.9956x; 1.0013x over previous
import functools

import jax
import jax.numpy as jnp
from jax import lax
from jax.experimental import pallas as pl
from jax.experimental.pallas import tpu as pltpu
from jax.experimental.pallas import tpu_sc as plsc

N = 50000
HID = 20
NEG = 0.2
ROWW = 24
ACCW = 24
NW = 32
EPW = 50176
EPAD = NW * EPW
CHUNK = 128
NCHUNK = EPW // CHUNK
NSLOT = 4
RACC = 51200
RPT = RACC // 16
DCH = 80
BIN = N
BLK = 1000
GRID = N // BLK
PAD0 = ROWW - HID - 1


def _elu(z):
    return jnp.where(z > 0, z, jnp.exp(jnp.minimum(z, 0.0)) - 1.0)


def _combine(p_ref, linp_ref, cbp_ref):
    num = p_ref[0, :, 0:HID] + p_ref[1, :, 0:HID]
    den = p_ref[0, :, HID:HID + 1] + p_ref[1, :, HID:HID + 1]
    conv = num / (den + 1e-30) + cbp_ref[...]
    return _elu(conv + linp_ref[...])


def _dense_common(h, W_ref, as_ref, ad_ref, lW_ref, lb_ref,
                  table_ref, dalpha_ref, amax_ref, lin_ref):
    xw = jnp.dot(h, W_ref[...], preferred_element_type=jnp.float32)
    asrc = jnp.dot(xw, as_ref[...], preferred_element_type=jnp.float32)
    adst = jnp.dot(xw, ad_ref[...], preferred_element_type=jnp.float32)
    table_ref[...] = jnp.concatenate(
        [xw, asrc, jnp.zeros((xw.shape[0], PAD0), jnp.float32)], axis=1)
    dalpha_ref[...] = adst
    lin_ref[...] = jnp.dot(h, lW_ref[...], preferred_element_type=jnp.float32) + lb_ref[...]
    m = jnp.max(asrc)

    @pl.when(pl.program_id(0) == 0)
    def _():
        amax_ref[...] = jnp.full((8, 128), -jnp.inf, jnp.float32)

    amax_ref[...] = jnp.maximum(amax_ref[...], m)


def _dense0_body(x_ref, W_ref, as_ref, ad_ref, lW_ref, lb_ref,
                 table_ref, dalpha_ref, amax_ref, lin_ref):
    _dense_common(x_ref[...], W_ref, as_ref, ad_ref, lW_ref, lb_ref,
                  table_ref, dalpha_ref, amax_ref, lin_ref)


def _densek_body(p_ref, linp_ref, cbp_ref, W_ref, as_ref, ad_ref, lW_ref, lb_ref,
                 table_ref, dalpha_ref, amax_ref, lin_ref):
    h = _combine(p_ref, linp_ref, cbp_ref)
    _dense_common(h, W_ref, as_ref, ad_ref, lW_ref, lb_ref,
                  table_ref, dalpha_ref, amax_ref, lin_ref)


def _mlp_body(p_ref, linp_ref, cbp_ref,
              w1, b1, w2, b2, w3, b3, w4, b4, out_ref):
    z = _combine(p_ref, linp_ref, cbp_ref)
    for w, b in ((w1, b1), (w2, b2), (w3, b3), (w4, b4)):
        z = _elu(jnp.dot(z, w[...], preferred_element_type=jnp.float32) + b[...])
    out_ref[...] = 1.0 / (1.0 + jnp.exp(-z))


def _full_spec(shape):
    nd = len(shape)
    return pl.BlockSpec(shape, lambda b, _n=nd: (0,) * _n)


_DENSE_OUT = [
    jax.ShapeDtypeStruct((N, ROWW), jnp.float32),
    jax.ShapeDtypeStruct((N, 1), jnp.float32),
    jax.ShapeDtypeStruct((8, 128), jnp.float32),
    jax.ShapeDtypeStruct((N, HID), jnp.float32),
]
_DENSE_OUT_SPECS = [
    pl.BlockSpec((BLK, ROWW), lambda b: (b, 0)),
    pl.BlockSpec((BLK, 1), lambda b: (b, 0)),
    pl.BlockSpec((8, 128), lambda b: (0, 0)),
    pl.BlockSpec((BLK, HID), lambda b: (b, 0)),
]


def _dense0(x, W, a_s, a_d, lW, lb):
    fin = x.shape[1]
    return pl.pallas_call(
        _dense0_body,
        grid=(GRID,),
        in_specs=[
            pl.BlockSpec((BLK, fin), lambda b: (b, 0)),
            _full_spec((fin, HID)),
            _full_spec((HID, 1)),
            _full_spec((HID, 1)),
            _full_spec((fin, HID)),
            _full_spec((1, HID)),
        ],
        out_specs=_DENSE_OUT_SPECS,
        out_shape=_DENSE_OUT,
    )(x, W, a_s, a_d, lW, lb)


def _densek(partial, lin_prev, cb_prev, W, a_s, a_d, lW, lb):
    return pl.pallas_call(
        _densek_body,
        grid=(GRID,),
        in_specs=[
            pl.BlockSpec((2, BLK, ACCW), lambda b: (0, b, 0)),
            pl.BlockSpec((BLK, HID), lambda b: (b, 0)),
            _full_spec((1, HID)),
            _full_spec((HID, HID)),
            _full_spec((HID, 1)),
            _full_spec((HID, 1)),
            _full_spec((HID, HID)),
            _full_spec((1, HID)),
        ],
        out_specs=_DENSE_OUT_SPECS,
        out_shape=_DENSE_OUT,
    )(partial, lin_prev, cb_prev, W, a_s, a_d, lW, lb)


def _mlp(partial, lin_prev, cb_prev, fWs, fbs):
    wspecs = []
    args = []
    for w, b in zip(fWs, fbs):
        wspecs += [_full_spec(w.shape), _full_spec((1,) + b.shape)]
        args += [w, b.reshape(1, -1)]
    return pl.pallas_call(
        _mlp_body,
        grid=(GRID,),
        in_specs=[
            pl.BlockSpec((2, BLK, ACCW), lambda b: (0, b, 0)),
            pl.BlockSpec((BLK, HID), lambda b: (b, 0)),
            _full_spec((1, HID)),
        ] + wspecs,
        out_specs=pl.BlockSpec((BLK, 1), lambda b: (b, 0)),
        out_shape=jax.ShapeDtypeStruct((N, 1), jnp.float32),
    )(partial, lin_prev, cb_prev, *args)


def _edge_body(table_h, eidx_h, dal_h, amx_h, out_h,
               acc,
               idx0, idx1, idx2, idx3,
               sdx0, sdx1, sdx2, sdx3,
               rows0, rows1, rows2, rows3,
               ad0, ad1, ad2, ad3,
               or0, or1, or2, or3,
               amx,
               g0, g1, g2, g3, a0, a1, a2, a3, s0, s1, s2, s3):
    cid = lax.axis_index("c")
    sid = lax.axis_index("s")
    wid = sid * 2 + cid

    idxs = (idx0, idx1, idx2, idx3)
    sdxs = (sdx0, sdx1, sdx2, sdx3)
    rowss = (rows0, rows1, rows2, rows3)
    ads = (ad0, ad1, ad2, ad3)
    ors = (or0, or1, or2, or3)
    gsem = (g0, g1, g2, g3)
    asem = (a0, a1, a2, a3)
    ssem = (s0, s1, s2, s3)

    pltpu.sync_copy(amx_h, amx)

    zero16 = jnp.zeros((16,), jnp.float32)

    for ob in ors:
        def zrow(r, c, _ob=ob):
            _ob[r, pl.ds(0, 16)] = zero16
            _ob[r, pl.ds(8, 16)] = zero16
            return c
        lax.fori_loop(0, CHUNK, zrow, 0)

    def zacc(i, c):
        pltpu.sync_copy(or0.at[pl.ds(0, DCH)],
                        acc.at[pl.ds(sid * RPT + i * DCH, DCH)])
        return c
    lax.fori_loop(0, RPT // DCH, zacc, 0)
    plsc.subcore_barrier()

    iota16 = lax.iota(jnp.int32, 16)
    amax_v = amx[...]
    col20 = jnp.full((16,), HID, jnp.int32)

    def issue(k, slot):
        pltpu.sync_copy(eidx_h.at[wid, k], idxs[slot])
        pltpu.async_copy(table_h.at[idxs[slot].at[0]], rowss[slot], gsem[slot])
        pltpu.async_copy(dal_h.at[idxs[slot].at[1]], ads[slot], asem[slot])

    def compute(slot):
        rows = rowss[slot]
        orows = ors[slot]
        adbuf = ads[slot]
        idxb = idxs[slot]
        sdx = sdxs[slot]
        es = []
        for g in range(CHUNK // 16):
            row_ids = g * 16 + iota16
            sdx[pl.ds(g * 16, 16)] = idxb[1, pl.ds(g * 16, 16)]
            a_d = adbuf[pl.ds(g * 16, 16)]
            t2 = amax_v + a_d
            shift = jnp.maximum(t2, NEG * t2)
            a_s = plsc.load_gather(rows, [row_ids, col20])
            t = a_s + a_d
            val = jnp.maximum(t, NEG * t)
            e = jnp.exp(val - shift)
            plsc.store_scatter(orows, [row_ids, col20], e)
            es.append(e)
        for g in range(CHUNK // 16):
            row_ids = g * 16 + iota16
            e = es[g]
            for f in range(HID):
                colf = jnp.full((16,), f, jnp.int32)
                v = plsc.load_gather(rows, [row_ids, colf])
                plsc.store_scatter(orows, [row_ids, colf], v * e)

    def drain_g(slot):
        pltpu.make_async_copy(table_h.at[pl.ds(0, CHUNK)],
                              rowss[slot], gsem[slot]).wait()

    def drain_a(slot):
        pltpu.make_async_copy(dal_h.at[pl.ds(0, CHUNK)],
                              ads[slot], asem[slot]).wait()

    def drain_s(slot):
        pltpu.make_async_copy(out_h.at[cid, pl.ds(0, CHUNK)],
                              ors[slot], ssem[slot]).wait()

    for k in range(NSLOT - 1):
        issue(k, k)

    def body(j, c):
        for u in range(NSLOT):
            k = NSLOT * j + u
            drain_g(u)
            drain_a(u)

            @pl.when(k + NSLOT - 1 < NCHUNK)
            def _(_u=u, _k=k):
                issue(_k + NSLOT - 1, (_u + NSLOT - 1) % NSLOT)

            @pl.when(j > 0)
            def _(_u=u):
                drain_s(_u)
            compute(u)
            pltpu.async_copy(ors[u], acc.at[sdxs[u]], ssem[u], add=True)
        return c
    lax.fori_loop(0, NCHUNK // NSLOT, body, 0)

    for u in range(NSLOT):
        drain_s(u)
    plsc.subcore_barrier()

    def dump(i, c):
        r0 = sid * RPT + i * DCH

        @pl.when(r0 < N)
        def _():
            pltpu.sync_copy(acc.at[pl.ds(r0, DCH)], or0.at[pl.ds(0, DCH)])
            pltpu.sync_copy(or0.at[pl.ds(0, DCH)], out_h.at[cid, pl.ds(r0, DCH)])
        return c
    lax.fori_loop(0, RPT // DCH, dump, 0)


def _edge_pass(table, eidx, dalpha, amax16):
    kern = pl.kernel(
        _edge_body,
        out_type=jax.ShapeDtypeStruct((2, N, ACCW), jnp.float32),
        mesh=plsc.VectorSubcoreMesh(core_axis_name="c", subcore_axis_name="s"),
        compiler_params=pltpu.CompilerParams(
            needs_layout_passes=False, use_tc_tiling_on_sc=False),
        scratch_types=(
            [pltpu.VMEM_SHARED((RACC, ACCW), jnp.float32)]
            + [pltpu.VMEM((2, CHUNK), jnp.int32) for _ in range(NSLOT)]
            + [pltpu.VMEM((CHUNK,), jnp.int32) for _ in range(NSLOT)]
            + [pltpu.VMEM((CHUNK, ROWW), jnp.float32) for _ in range(NSLOT)]
            + [pltpu.VMEM((CHUNK,), jnp.float32) for _ in range(NSLOT)]
            + [pltpu.VMEM((CHUNK, ACCW), jnp.float32) for _ in range(NSLOT)]
            + [pltpu.VMEM((16,), jnp.float32)]
            + [pltpu.SemaphoreType.DMA for _ in range(3 * NSLOT)]
        ),
    )
    return kern(table, eidx, dalpha, amax16)


def kernel(x, edge_index, params):
    src = edge_index[0]
    dst = edge_index[1]
    pad = EPAD - src.shape[0]
    srcs = jnp.concatenate([src, jnp.zeros((pad,), jnp.int32)])
    dsts = jnp.concatenate([dst, jnp.full((pad,), BIN, jnp.int32)])
    eidx = jnp.stack([srcs.reshape(NW, NCHUNK, CHUNK),
                      dsts.reshape(NW, NCHUNK, CHUNK)], axis=2)

    partial = None
    lin = None
    for i in range(4):
        W = params["cW"][i]
        a_s = params["cas"][i].reshape(HID, 1)
        a_d = params["cad"][i].reshape(HID, 1)
        lW = params["lW"][i]
        lb = params["lb"][i].reshape(1, HID)
        if i == 0:
            table, dalpha, amax, lin = _dense0(x, W, a_s, a_d, lW, lb)
        else:
            cbp = params["cb"][i - 1].reshape(1, HID)
            table, dalpha, amax, lin = _densek(
                partial, lin, cbp, W, a_s, a_d, lW, lb)
        partial = _edge_pass(table, eidx, dalpha.reshape(N), amax[0, 0:16])

    out = _mlp(partial, lin, params["cb"][3].reshape(1, HID),
               params["fW"], params["fb"])
    return out.reshape(N)


# speedup vs baseline: 106.0195x; 1.9635x over previous
import functools

import jax
import jax.numpy as jnp
from jax import lax
from jax.experimental import pallas as pl
from jax.experimental.pallas import tpu as pltpu
from jax.experimental.pallas import tpu_sc as plsc

N = 50000
HID = 20
NEG = 0.2
ROWW = 24
ACCW = 24
NW = 32
EPW = 50176
EPAD = NW * EPW
CHUNK = 128
NCHUNK = EPW // CHUNK
NSLOT = 4
RACC = 51200
RPT = RACC // 16
DCH = 80
BIN = N
BLK = 1000
GRID = N // BLK
PAD0 = ROWW - HID - 1


def _elu(z):
    return jnp.where(z > 0, z, jnp.exp(jnp.minimum(z, 0.0)) - 1.0)


def _combine(p_ref, linp_ref, cbp_ref):
    num = p_ref[0, :, 0:HID] + p_ref[1, :, 0:HID]
    den = p_ref[0, :, HID:HID + 1] + p_ref[1, :, HID:HID + 1]
    conv = num / (den + 1e-30) + cbp_ref[...]
    return _elu(conv + linp_ref[...])


def _dense_common(h, W_ref, as_ref, ad_ref, lW_ref, lb_ref,
                  table_ref, dalpha_ref, amax_ref, lin_ref):
    xw = jnp.dot(h, W_ref[...], preferred_element_type=jnp.float32)
    asrc = jnp.dot(xw, as_ref[...], preferred_element_type=jnp.float32)
    adst = jnp.dot(xw, ad_ref[...], preferred_element_type=jnp.float32)
    table_ref[...] = jnp.concatenate(
        [xw, asrc, jnp.zeros((xw.shape[0], PAD0), jnp.float32)], axis=1)
    dalpha_ref[...] = adst
    lin_ref[...] = jnp.dot(h, lW_ref[...], preferred_element_type=jnp.float32) + lb_ref[...]
    m = jnp.max(asrc)

    @pl.when(pl.program_id(0) == 0)
    def _():
        amax_ref[...] = jnp.full((8, 128), -jnp.inf, jnp.float32)

    amax_ref[...] = jnp.maximum(amax_ref[...], m)


def _dense0_body(x_ref, W_ref, as_ref, ad_ref, lW_ref, lb_ref,
                 table_ref, dalpha_ref, amax_ref, lin_ref):
    _dense_common(x_ref[...], W_ref, as_ref, ad_ref, lW_ref, lb_ref,
                  table_ref, dalpha_ref, amax_ref, lin_ref)


def _densek_body(p_ref, linp_ref, cbp_ref, W_ref, as_ref, ad_ref, lW_ref, lb_ref,
                 table_ref, dalpha_ref, amax_ref, lin_ref):
    h = _combine(p_ref, linp_ref, cbp_ref)
    _dense_common(h, W_ref, as_ref, ad_ref, lW_ref, lb_ref,
                  table_ref, dalpha_ref, amax_ref, lin_ref)


def _mlp_body(p_ref, linp_ref, cbp_ref,
              w1, b1, w2, b2, w3, b3, w4, b4, out_ref):
    z = _combine(p_ref, linp_ref, cbp_ref)
    for w, b in ((w1, b1), (w2, b2), (w3, b3), (w4, b4)):
        z = _elu(jnp.dot(z, w[...], preferred_element_type=jnp.float32) + b[...])
    out_ref[...] = 1.0 / (1.0 + jnp.exp(-z))


def _full_spec(shape):
    nd = len(shape)
    return pl.BlockSpec(shape, lambda b, _n=nd: (0,) * _n)


_DENSE_OUT = [
    jax.ShapeDtypeStruct((N, ROWW), jnp.float32),
    jax.ShapeDtypeStruct((N, 1), jnp.float32),
    jax.ShapeDtypeStruct((8, 128), jnp.float32),
    jax.ShapeDtypeStruct((N, HID), jnp.float32),
]
_DENSE_OUT_SPECS = [
    pl.BlockSpec((BLK, ROWW), lambda b: (b, 0)),
    pl.BlockSpec((BLK, 1), lambda b: (b, 0)),
    pl.BlockSpec((8, 128), lambda b: (0, 0)),
    pl.BlockSpec((BLK, HID), lambda b: (b, 0)),
]


def _dense0(x, W, a_s, a_d, lW, lb):
    fin = x.shape[1]
    return pl.pallas_call(
        _dense0_body,
        grid=(GRID,),
        in_specs=[
            pl.BlockSpec((BLK, fin), lambda b: (b, 0)),
            _full_spec((fin, HID)),
            _full_spec((HID, 1)),
            _full_spec((HID, 1)),
            _full_spec((fin, HID)),
            _full_spec((1, HID)),
        ],
        out_specs=_DENSE_OUT_SPECS,
        out_shape=_DENSE_OUT,
    )(x, W, a_s, a_d, lW, lb)


def _densek(partial, lin_prev, cb_prev, W, a_s, a_d, lW, lb):
    return pl.pallas_call(
        _densek_body,
        grid=(GRID,),
        in_specs=[
            pl.BlockSpec((2, BLK, ACCW), lambda b: (0, b, 0)),
            pl.BlockSpec((BLK, HID), lambda b: (b, 0)),
            _full_spec((1, HID)),
            _full_spec((HID, HID)),
            _full_spec((HID, 1)),
            _full_spec((HID, 1)),
            _full_spec((HID, HID)),
            _full_spec((1, HID)),
        ],
        out_specs=_DENSE_OUT_SPECS,
        out_shape=_DENSE_OUT,
    )(partial, lin_prev, cb_prev, W, a_s, a_d, lW, lb)


def _mlp(partial, lin_prev, cb_prev, fWs, fbs):
    wspecs = []
    args = []
    for w, b in zip(fWs, fbs):
        wspecs += [_full_spec(w.shape), _full_spec((1,) + b.shape)]
        args += [w, b.reshape(1, -1)]
    return pl.pallas_call(
        _mlp_body,
        grid=(GRID,),
        in_specs=[
            pl.BlockSpec((2, BLK, ACCW), lambda b: (0, b, 0)),
            pl.BlockSpec((BLK, HID), lambda b: (b, 0)),
            _full_spec((1, HID)),
        ] + wspecs,
        out_specs=pl.BlockSpec((BLK, 1), lambda b: (b, 0)),
        out_shape=jax.ShapeDtypeStruct((N, 1), jnp.float32),
    )(partial, lin_prev, cb_prev, *args)


def _edge_body(table_h, eidx_h, dal_h, amx_h, out_h,
               acc,
               idx0, idx1, idx2, idx3,
               sdx0, sdx1, sdx2, sdx3,
               rows0, rows1, rows2, rows3,
               ad0, ad1, ad2, ad3,
               or0, or1, or2, or3,
               amx, ebuf,
               g0, g1, g2, g3, a0, a1, a2, a3, s0, s1, s2, s3):
    cid = lax.axis_index("c")
    sid = lax.axis_index("s")
    wid = sid * 2 + cid

    idxs = (idx0, idx1, idx2, idx3)
    sdxs = (sdx0, sdx1, sdx2, sdx3)
    rowss = (rows0, rows1, rows2, rows3)
    ads = (ad0, ad1, ad2, ad3)
    ors = (or0, or1, or2, or3)
    gsem = (g0, g1, g2, g3)
    asem = (a0, a1, a2, a3)
    ssem = (s0, s1, s2, s3)

    pltpu.sync_copy(amx_h, amx)

    zero16 = jnp.zeros((16,), jnp.float32)

    for ob in ors:
        def zrow(r, c, _ob=ob):
            _ob[r, pl.ds(0, 16)] = zero16
            _ob[r, pl.ds(8, 16)] = zero16
            return c
        lax.fori_loop(0, CHUNK, zrow, 0)

    def zacc(i, c):
        pltpu.sync_copy(or0.at[pl.ds(0, DCH)],
                        acc.at[pl.ds(sid * RPT + i * DCH, DCH)])
        return c
    lax.fori_loop(0, RPT // DCH, zacc, 0)
    plsc.subcore_barrier()

    iota16 = lax.iota(jnp.int32, 16)
    amax_v = amx[...]
    col20 = jnp.full((16,), HID, jnp.int32)

    def issue(k, slot):
        pltpu.sync_copy(eidx_h.at[wid, k], idxs[slot])
        pltpu.async_copy(table_h.at[idxs[slot].at[0]], rowss[slot], gsem[slot])
        pltpu.async_copy(dal_h.at[idxs[slot].at[1]], ads[slot], asem[slot])

    def compute(slot):
        rows = rowss[slot]
        orows = ors[slot]
        adbuf = ads[slot]
        idxb = idxs[slot]
        sdx = sdxs[slot]
        ones16 = jnp.ones((16,), jnp.float32)
        es = []
        for g in range(CHUNK // 16):
            row_ids = g * 16 + iota16
            sdx[pl.ds(g * 16, 16)] = idxb[1, pl.ds(g * 16, 16)]
            a_d = adbuf[pl.ds(g * 16, 16)]
            t2 = amax_v + a_d
            shift = jnp.maximum(t2, NEG * t2)
            a_s = plsc.load_gather(rows, [row_ids, col20])
            t = a_s + a_d
            val = jnp.maximum(t, NEG * t)
            e = jnp.exp(val - shift)
            plsc.store_scatter(rows, [row_ids, col20], ones16)
            es.append(e)
        for g in range(CHUNK // 16):
            ev = es[g]
            for j in range(16):
                r = g * 16 + j
                ej = ev[j]
                orows[r, pl.ds(0, 16)] = rows[r, pl.ds(0, 16)] * ej
                orows[r, pl.ds(8, 16)] = rows[r, pl.ds(8, 16)] * ej

    def drain_g(slot):
        pltpu.make_async_copy(table_h.at[pl.ds(0, CHUNK)],
                              rowss[slot], gsem[slot]).wait()

    def drain_a(slot):
        pltpu.make_async_copy(dal_h.at[pl.ds(0, CHUNK)],
                              ads[slot], asem[slot]).wait()

    def drain_s(slot):
        pltpu.make_async_copy(out_h.at[cid, pl.ds(0, CHUNK)],
                              ors[slot], ssem[slot]).wait()

    for k in range(NSLOT - 1):
        issue(k, k)

    def body(j, c):
        for u in range(NSLOT):
            k = NSLOT * j + u
            drain_g(u)
            drain_a(u)

            @pl.when(k + NSLOT - 1 < NCHUNK)
            def _(_u=u, _k=k):
                issue(_k + NSLOT - 1, (_u + NSLOT - 1) % NSLOT)

            @pl.when(j > 0)
            def _(_u=u):
                drain_s(_u)
            compute(u)
            pltpu.async_copy(ors[u], acc.at[sdxs[u]], ssem[u], add=True)
        return c
    lax.fori_loop(0, NCHUNK // NSLOT, body, 0)

    for u in range(NSLOT):
        drain_s(u)
    plsc.subcore_barrier()

    def dump(i, c):
        r0 = sid * RPT + i * DCH

        @pl.when(r0 < N)
        def _():
            pltpu.sync_copy(acc.at[pl.ds(r0, DCH)], or0.at[pl.ds(0, DCH)])
            pltpu.sync_copy(or0.at[pl.ds(0, DCH)], out_h.at[cid, pl.ds(r0, DCH)])
        return c
    lax.fori_loop(0, RPT // DCH, dump, 0)


def _edge_pass(table, eidx, dalpha, amax16):
    kern = pl.kernel(
        _edge_body,
        out_type=jax.ShapeDtypeStruct((2, N, ACCW), jnp.float32),
        mesh=plsc.VectorSubcoreMesh(core_axis_name="c", subcore_axis_name="s"),
        compiler_params=pltpu.CompilerParams(
            needs_layout_passes=False, use_tc_tiling_on_sc=False),
        scratch_types=(
            [pltpu.VMEM_SHARED((RACC, ACCW), jnp.float32)]
            + [pltpu.VMEM((2, CHUNK), jnp.int32) for _ in range(NSLOT)]
            + [pltpu.VMEM((CHUNK,), jnp.int32) for _ in range(NSLOT)]
            + [pltpu.VMEM((CHUNK, ROWW), jnp.float32) for _ in range(NSLOT)]
            + [pltpu.VMEM((CHUNK,), jnp.float32) for _ in range(NSLOT)]
            + [pltpu.VMEM((CHUNK, ACCW), jnp.float32) for _ in range(NSLOT)]
            + [pltpu.VMEM((16,), jnp.float32)]
            + [pltpu.VMEM((CHUNK,), jnp.float32)]
            + [pltpu.SemaphoreType.DMA for _ in range(3 * NSLOT)]
        ),
    )
    return kern(table, eidx, dalpha, amax16)


def kernel(x, edge_index, params):
    src = edge_index[0]
    dst = edge_index[1]
    pad = EPAD - src.shape[0]
    srcs = jnp.concatenate([src, jnp.zeros((pad,), jnp.int32)])
    dsts = jnp.concatenate([dst, jnp.full((pad,), BIN, jnp.int32)])
    eidx = jnp.stack([srcs.reshape(NW, NCHUNK, CHUNK),
                      dsts.reshape(NW, NCHUNK, CHUNK)], axis=2)

    partial = None
    lin = None
    for i in range(4):
        W = params["cW"][i]
        a_s = params["cas"][i].reshape(HID, 1)
        a_d = params["cad"][i].reshape(HID, 1)
        lW = params["lW"][i]
        lb = params["lb"][i].reshape(1, HID)
        if i == 0:
            table, dalpha, amax, lin = _dense0(x, W, a_s, a_d, lW, lb)
        else:
            cbp = params["cb"][i - 1].reshape(1, HID)
            table, dalpha, amax, lin = _densek(
                partial, lin, cbp, W, a_s, a_d, lW, lb)
        partial = _edge_pass(table, eidx, dalpha.reshape(N), amax[0, 0:16])

    out = _mlp(partial, lin, params["cb"][3].reshape(1, HID),
               params["fW"], params["fb"])
    return out.reshape(N)


# speedup vs baseline: 135.7723x; 1.2806x over previous
import functools

import jax
import jax.numpy as jnp
from jax import lax
from jax.experimental import pallas as pl
from jax.experimental.pallas import tpu as pltpu
from jax.experimental.pallas import tpu_sc as plsc

N = 50000
HID = 20
NEG = 0.2
ROWW = 24
ACCW = 24
NW = 32
EPW = 50176
EPAD = NW * EPW
CHUNK = 128
NCHUNK = EPW // CHUNK
NSLOT = 4
CPB = 56
BAT = CPB * CHUNK
RACC = 51200
RPT = RACC // 16
DCH = 80
BIN = N
BLK = 1000
GRID = N // BLK
PAD0 = ROWW - HID - 1


def _elu(z):
    return jnp.where(z > 0, z, jnp.exp(jnp.minimum(z, 0.0)) - 1.0)


def _combine(p_ref, linp_ref, cbp_ref):
    num = p_ref[0, :, 0:HID] + p_ref[1, :, 0:HID]
    den = p_ref[0, :, HID:HID + 1] + p_ref[1, :, HID:HID + 1]
    conv = num / (den + 1e-30) + cbp_ref[...]
    return _elu(conv + linp_ref[...])


def _dense_common(h, W_ref, as_ref, ad_ref, lW_ref, lb_ref,
                  table_ref, dalpha_ref, amax_ref, lin_ref):
    xw = jnp.dot(h, W_ref[...], preferred_element_type=jnp.float32)
    asrc = jnp.dot(xw, as_ref[...], preferred_element_type=jnp.float32)
    adst = jnp.dot(xw, ad_ref[...], preferred_element_type=jnp.float32)
    table_ref[...] = jnp.concatenate(
        [xw, asrc, jnp.zeros((xw.shape[0], PAD0), jnp.float32)], axis=1)
    dalpha_ref[...] = adst
    lin_ref[...] = jnp.dot(h, lW_ref[...], preferred_element_type=jnp.float32) + lb_ref[...]
    m = jnp.max(asrc)

    @pl.when(pl.program_id(0) == 0)
    def _():
        amax_ref[...] = jnp.full((8, 128), -jnp.inf, jnp.float32)

    amax_ref[...] = jnp.maximum(amax_ref[...], m)


def _dense0_body(x_ref, W_ref, as_ref, ad_ref, lW_ref, lb_ref,
                 table_ref, dalpha_ref, amax_ref, lin_ref):
    _dense_common(x_ref[...], W_ref, as_ref, ad_ref, lW_ref, lb_ref,
                  table_ref, dalpha_ref, amax_ref, lin_ref)


def _densek_body(p_ref, linp_ref, cbp_ref, W_ref, as_ref, ad_ref, lW_ref, lb_ref,
                 table_ref, dalpha_ref, amax_ref, lin_ref):
    h = _combine(p_ref, linp_ref, cbp_ref)
    _dense_common(h, W_ref, as_ref, ad_ref, lW_ref, lb_ref,
                  table_ref, dalpha_ref, amax_ref, lin_ref)


def _mlp_body(p_ref, linp_ref, cbp_ref,
              w1, b1, w2, b2, w3, b3, w4, b4, out_ref):
    z = _combine(p_ref, linp_ref, cbp_ref)
    for w, b in ((w1, b1), (w2, b2), (w3, b3), (w4, b4)):
        z = _elu(jnp.dot(z, w[...], preferred_element_type=jnp.float32) + b[...])
    out_ref[...] = 1.0 / (1.0 + jnp.exp(-z))


def _full_spec(shape):
    nd = len(shape)
    return pl.BlockSpec(shape, lambda b, _n=nd: (0,) * _n)


_DENSE_OUT = [
    jax.ShapeDtypeStruct((N, ROWW), jnp.float32),
    jax.ShapeDtypeStruct((N, 1), jnp.float32),
    jax.ShapeDtypeStruct((8, 128), jnp.float32),
    jax.ShapeDtypeStruct((N, HID), jnp.float32),
]
_DENSE_OUT_SPECS = [
    pl.BlockSpec((BLK, ROWW), lambda b: (b, 0)),
    pl.BlockSpec((BLK, 1), lambda b: (b, 0)),
    pl.BlockSpec((8, 128), lambda b: (0, 0)),
    pl.BlockSpec((BLK, HID), lambda b: (b, 0)),
]


def _dense0(x, W, a_s, a_d, lW, lb):
    fin = x.shape[1]
    return pl.pallas_call(
        _dense0_body,
        grid=(GRID,),
        in_specs=[
            pl.BlockSpec((BLK, fin), lambda b: (b, 0)),
            _full_spec((fin, HID)),
            _full_spec((HID, 1)),
            _full_spec((HID, 1)),
            _full_spec((fin, HID)),
            _full_spec((1, HID)),
        ],
        out_specs=_DENSE_OUT_SPECS,
        out_shape=_DENSE_OUT,
    )(x, W, a_s, a_d, lW, lb)


def _densek(partial, lin_prev, cb_prev, W, a_s, a_d, lW, lb):
    return pl.pallas_call(
        _densek_body,
        grid=(GRID,),
        in_specs=[
            pl.BlockSpec((2, BLK, ACCW), lambda b: (0, b, 0)),
            pl.BlockSpec((BLK, HID), lambda b: (b, 0)),
            _full_spec((1, HID)),
            _full_spec((HID, HID)),
            _full_spec((HID, 1)),
            _full_spec((HID, 1)),
            _full_spec((HID, HID)),
            _full_spec((1, HID)),
        ],
        out_specs=_DENSE_OUT_SPECS,
        out_shape=_DENSE_OUT,
    )(partial, lin_prev, cb_prev, W, a_s, a_d, lW, lb)


def _mlp(partial, lin_prev, cb_prev, fWs, fbs):
    wspecs = []
    args = []
    for w, b in zip(fWs, fbs):
        wspecs += [_full_spec(w.shape), _full_spec((1,) + b.shape)]
        args += [w, b.reshape(1, -1)]
    return pl.pallas_call(
        _mlp_body,
        grid=(GRID,),
        in_specs=[
            pl.BlockSpec((2, BLK, ACCW), lambda b: (0, b, 0)),
            pl.BlockSpec((BLK, HID), lambda b: (b, 0)),
            _full_spec((1, HID)),
        ] + wspecs,
        out_specs=pl.BlockSpec((BLK, 1), lambda b: (b, 0)),
        out_shape=jax.ShapeDtypeStruct((N, 1), jnp.float32),
    )(partial, lin_prev, cb_prev, *args)


def _edge_body(table_h, eidx_h, dal_h, amx_h, out_h,
               acc,
               ibuf,
               sdx0, sdx1, sdx2, sdx3,
               rows0, rows1, rows2, rows3,
               ad0, ad1, ad2, ad3,
               or0, or1, or2, or3,
               amx, ebuf,
               g0, g1, g2, g3, a0, a1, a2, a3, s0, s1, s2, s3):
    cid = lax.axis_index("c")
    sid = lax.axis_index("s")
    wid = sid * 2 + cid

    sdxs = (sdx0, sdx1, sdx2, sdx3)
    rowss = (rows0, rows1, rows2, rows3)
    ads = (ad0, ad1, ad2, ad3)
    ors = (or0, or1, or2, or3)
    gsem = (g0, g1, g2, g3)
    asem = (a0, a1, a2, a3)
    ssem = (s0, s1, s2, s3)

    pltpu.sync_copy(amx_h, amx)

    zero16 = jnp.zeros((16,), jnp.float32)

    for ob in ors:
        def zrow(r, c, _ob=ob):
            _ob[r, pl.ds(0, 16)] = zero16
            _ob[r, pl.ds(8, 16)] = zero16
            return c
        lax.fori_loop(0, CHUNK, zrow, 0)

    def zacc(i, c):
        pltpu.sync_copy(or0.at[pl.ds(0, DCH)],
                        acc.at[pl.ds(sid * RPT + i * DCH, DCH)])
        return c
    lax.fori_loop(0, RPT // DCH, zacc, 0)
    plsc.subcore_barrier()

    iota16 = lax.iota(jnp.int32, 16)
    amax_v = amx[...]
    col20 = jnp.full((16,), HID, jnp.int32)

    def issue(off, slot):
        pltpu.async_copy(table_h.at[ibuf.at[0, pl.ds(off, CHUNK)]],
                         rowss[slot], gsem[slot])
        pltpu.async_copy(dal_h.at[ibuf.at[1, pl.ds(off, CHUNK)]],
                         ads[slot], asem[slot])

    def compute(slot, off):
        rows = rowss[slot]
        orows = ors[slot]
        adbuf = ads[slot]
        sdx = sdxs[slot]
        ones16 = jnp.ones((16,), jnp.float32)
        es = []
        for g in range(CHUNK // 16):
            row_ids = g * 16 + iota16
            sdx[pl.ds(g * 16, 16)] = ibuf[1, pl.ds(off + g * 16, 16)]
            a_d = adbuf[pl.ds(g * 16, 16)]
            t2 = amax_v + a_d
            shift = jnp.maximum(t2, NEG * t2)
            a_s = plsc.load_gather(rows, [row_ids, col20])
            t = a_s + a_d
            val = jnp.maximum(t, NEG * t)
            e = jnp.exp(val - shift)
            plsc.store_scatter(rows, [row_ids, col20], ones16)
            es.append(e)
        for g in range(CHUNK // 16):
            ev = es[g]
            for j in range(16):
                r = g * 16 + j
                ej = ev[j]
                orows[r, pl.ds(0, 16)] = rows[r, pl.ds(0, 16)] * ej
                orows[r, pl.ds(8, 16)] = rows[r, pl.ds(8, 16)] * ej

    def drain_g(slot):
        pltpu.make_async_copy(table_h.at[pl.ds(0, CHUNK)],
                              rowss[slot], gsem[slot]).wait()

    def drain_a(slot):
        pltpu.make_async_copy(dal_h.at[pl.ds(0, CHUNK)],
                              ads[slot], asem[slot]).wait()

    def drain_s(slot):
        pltpu.make_async_copy(out_h.at[cid, pl.ds(0, CHUNK)],
                              ors[slot], ssem[slot]).wait()

    def batch(b, c0):
        pltpu.sync_copy(eidx_h.at[wid, 0, pl.ds(b * BAT, BAT)], ibuf.at[0])
        pltpu.sync_copy(eidx_h.at[wid, 1, pl.ds(b * BAT, BAT)], ibuf.at[1])
        for k in range(NSLOT - 1):
            issue(k * CHUNK, k)

        def body(j, c):
            for u in range(NSLOT):
                k = NSLOT * j + u
                drain_g(u)
                drain_a(u)

                @pl.when(k + NSLOT - 1 < CPB)
                def _(_u=u, _k=k):
                    issue((_k + NSLOT - 1) * CHUNK, (_u + NSLOT - 1) % NSLOT)

                @pl.when(jnp.logical_or(b > 0, j > 0))
                def _(_u=u):
                    drain_s(_u)
                compute(u, k * CHUNK)
                pltpu.async_copy(ors[u], acc.at[sdxs[u]], ssem[u], add=True)
            return c
        lax.fori_loop(0, CPB // NSLOT, body, 0)
        return c0
    lax.fori_loop(0, NCHUNK // CPB, batch, 0)

    for u in range(NSLOT):
        drain_s(u)
    plsc.subcore_barrier()

    def dump(i, c):
        r0 = sid * RPT + i * DCH

        @pl.when(r0 < N)
        def _():
            pltpu.sync_copy(acc.at[pl.ds(r0, DCH)], or0.at[pl.ds(0, DCH)])
            pltpu.sync_copy(or0.at[pl.ds(0, DCH)], out_h.at[cid, pl.ds(r0, DCH)])
        return c
    lax.fori_loop(0, RPT // DCH, dump, 0)


def _edge_pass(table, eidx, dalpha, amax16):
    kern = pl.kernel(
        _edge_body,
        out_type=jax.ShapeDtypeStruct((2, N, ACCW), jnp.float32),
        mesh=plsc.VectorSubcoreMesh(core_axis_name="c", subcore_axis_name="s"),
        compiler_params=pltpu.CompilerParams(
            needs_layout_passes=False, use_tc_tiling_on_sc=False),
        scratch_types=(
            [pltpu.VMEM_SHARED((RACC, ACCW), jnp.float32)]
            + [pltpu.VMEM((2, BAT), jnp.int32)]
            + [pltpu.VMEM((CHUNK,), jnp.int32) for _ in range(NSLOT)]
            + [pltpu.VMEM((CHUNK, ROWW), jnp.float32) for _ in range(NSLOT)]
            + [pltpu.VMEM((CHUNK,), jnp.float32) for _ in range(NSLOT)]
            + [pltpu.VMEM((CHUNK, ACCW), jnp.float32) for _ in range(NSLOT)]
            + [pltpu.VMEM((16,), jnp.float32)]
            + [pltpu.VMEM((CHUNK,), jnp.float32)]
            + [pltpu.SemaphoreType.DMA for _ in range(3 * NSLOT)]
        ),
    )
    return kern(table, eidx, dalpha, amax16)


def kernel(x, edge_index, params):
    src = edge_index[0]
    dst = edge_index[1]
    pad = EPAD - src.shape[0]
    srcs = jnp.concatenate([src, jnp.zeros((pad,), jnp.int32)])
    dsts = jnp.concatenate([dst, jnp.full((pad,), BIN, jnp.int32)])
    eidx = jnp.stack([srcs.reshape(NW, EPW), dsts.reshape(NW, EPW)], axis=1)

    partial = None
    lin = None
    for i in range(4):
        W = params["cW"][i]
        a_s = params["cas"][i].reshape(HID, 1)
        a_d = params["cad"][i].reshape(HID, 1)
        lW = params["lW"][i]
        lb = params["lb"][i].reshape(1, HID)
        if i == 0:
            table, dalpha, amax, lin = _dense0(x, W, a_s, a_d, lW, lb)
        else:
            cbp = params["cb"][i - 1].reshape(1, HID)
            table, dalpha, amax, lin = _densek(
                partial, lin, cbp, W, a_s, a_d, lW, lb)
        partial = _edge_pass(table, eidx, dalpha.reshape(N), amax[0, 0:16])

    out = _mlp(partial, lin, params["cb"][3].reshape(1, HID),
               params["fW"], params["fb"])
    return out.reshape(N)
